# Initial kernel scaffold; baseline (speedup 1.0000x reference)
import jax
import jax.numpy as jnp
from jax.experimental import pallas as pl


def kernel(x, edge_index, W1_l, b1_l, W1_r, W2_l, b2_l, W2_r):
    raise NotImplementedError("write your pallas kernel here")


# speedup vs baseline: 6.2953x; 6.2953x over previous
import functools

import jax
import jax.numpy as jnp
from jax import lax
from jax.experimental import pallas as pl
from jax.experimental.pallas import tpu as pltpu
from jax.experimental.pallas import tpu_sc as plsc

N = 10000
F = 128
C = 16
E = 320000
NPAD = 10112
NSC = 2
NTEC = 16
NTILES = NSC * NTEC
CHUNK = 128
NCHUNK = 79
EPT = NCHUNK * CHUNK
ROWS_PT = NPAD // NTEC

BLK = 400
GRID = N // BLK


def _make_sc_scatter(width, tc_tiling=True):
  mesh = plsc.VectorSubcoreMesh(core_axis_name="c", subcore_axis_name="s")

  @functools.partial(
      pl.kernel,
      out_type=jax.ShapeDtypeStruct((NSC, NPAD, width), jnp.float32),
      mesh=mesh,
      compiler_params=pltpu.CompilerParams(use_tc_tiling_on_sc=tc_tiling),
      scratch_types=[
          pltpu.VMEM((NCHUNK, CHUNK), jnp.int32),
          pltpu.VMEM((NCHUNK, CHUNK), jnp.int32),
          pltpu.VMEM((CHUNK, width), jnp.float32),
          pltpu.VMEM_SHARED((NPAD, width), jnp.float32),
          pltpu.SemaphoreType.DMA,
      ],
  )
  def sc_scatter(table, src3, dst3, zeros, out, src_v, dst_v, rows_v, acc, sem):
    c = lax.axis_index("c")
    s = lax.axis_index("s")
    tile = c * NTEC + s
    pltpu.sync_copy(zeros.at[pl.ds(s * ROWS_PT, ROWS_PT)],
                    acc.at[pl.ds(s * ROWS_PT, ROWS_PT)])
    pltpu.sync_copy(src3.at[tile], src_v)
    pltpu.sync_copy(dst3.at[tile], dst_v)
    plsc.subcore_barrier()

    def body(j, carry):
      pltpu.async_copy(table.at[src_v.at[j]], rows_v, sem).wait()
      pltpu.sync_copy(rows_v, acc.at[dst_v.at[j]], add=True)
      return carry

    lax.fori_loop(0, NCHUNK, body, 0)
    plsc.subcore_barrier()
    pltpu.sync_copy(acc.at[pl.ds(s * ROWS_PT, ROWS_PT)],
                    out.at[c, pl.ds(s * ROWS_PT, ROWS_PT)])

  return sc_scatter


_sc_scatter_f = _make_sc_scatter(F)
_sc_scatter_c = _make_sc_scatter(C, tc_tiling=False)


def _dense1(x, wl, wr, b1):
  def body(x_ref, wl_ref, wr_ref, b_ref, xl_ref, xd_ref):
    xb = x_ref[...]
    xl_ref[...] = jnp.dot(xb, wl_ref[...], preferred_element_type=jnp.float32)
    xd_ref[...] = jnp.dot(xb, wr_ref[...], preferred_element_type=jnp.float32) + b_ref[...]

  return pl.pallas_call(
      body,
      grid=(GRID,),
      in_specs=[
          pl.BlockSpec((BLK, F), lambda i: (i, 0)),
          pl.BlockSpec((F, F), lambda i: (0, 0)),
          pl.BlockSpec((F, F), lambda i: (0, 0)),
          pl.BlockSpec((1, F), lambda i: (0, 0)),
      ],
      out_specs=[pl.BlockSpec((BLK, F), lambda i: (i, 0))] * 2,
      out_shape=[jax.ShapeDtypeStruct((N, F), jnp.float32)] * 2,
  )(x, wl, wr, b1.reshape(1, F))


def _dense2(p, xd, w2l, w2r, b2):
  def body(p_ref, xd_ref, wl_ref, wr_ref, b_ref, hl_ref, hd_ref):
    h = jnp.maximum(p_ref[0] + p_ref[1] + xd_ref[...], 0.0)
    hl_ref[...] = jnp.dot(h, wl_ref[...], preferred_element_type=jnp.float32)
    hd_ref[...] = jnp.dot(h, wr_ref[...], preferred_element_type=jnp.float32) + b_ref[...]

  return pl.pallas_call(
      body,
      grid=(GRID,),
      in_specs=[
          pl.BlockSpec((NSC, BLK, F), lambda i: (0, i, 0)),
          pl.BlockSpec((BLK, F), lambda i: (i, 0)),
          pl.BlockSpec((F, C), lambda i: (0, 0)),
          pl.BlockSpec((F, C), lambda i: (0, 0)),
          pl.BlockSpec((1, C), lambda i: (0, 0)),
      ],
      out_specs=[pl.BlockSpec((BLK, C), lambda i: (i, 0))] * 2,
      out_shape=[jax.ShapeDtypeStruct((N, C), jnp.float32)] * 2,
  )(p, xd, w2l, w2r, b2.reshape(1, C))


def _final(q, hd):
  def body(q_ref, hd_ref, o_ref):
    o_ref[...] = q_ref[0] + q_ref[1] + hd_ref[...]

  return pl.pallas_call(
      body,
      grid=(GRID,),
      in_specs=[
          pl.BlockSpec((NSC, BLK, C), lambda i: (0, i, 0)),
          pl.BlockSpec((BLK, C), lambda i: (i, 0)),
      ],
      out_specs=pl.BlockSpec((BLK, C), lambda i: (i, 0)),
      out_shape=jax.ShapeDtypeStruct((N, C), jnp.float32),
  )(q, hd)


def kernel(x, edge_index, W1_l, b1_l, W1_r, W2_l, b2_l, W2_r):
  src = edge_index[0]
  dst = edge_index[1]
  pad = NTILES * EPT - E
  src3 = jnp.concatenate([src, jnp.zeros((pad,), jnp.int32)]).reshape(NTILES, NCHUNK, CHUNK)
  dst3 = jnp.concatenate([dst, jnp.full((pad,), N, jnp.int32)]).reshape(NTILES, NCHUNK, CHUNK)
  zeros_f = jnp.zeros((NPAD, F), jnp.float32)
  zeros_c = jnp.zeros((NPAD, C), jnp.float32)

  xl, xd = _dense1(x, W1_l, W1_r, b1_l)
  p = _sc_scatter_f(xl, src3, dst3, zeros_f)
  hl, hd = _dense2(p, xd, W2_l, W2_r, b2_l)
  q = _sc_scatter_c(hl, src3, dst3, zeros_c)
  return _final(q, hd)
